# Initial kernel scaffold; baseline (speedup 1.0000x reference)
import jax
import jax.numpy as jnp
from jax.experimental import pallas as pl


def kernel(x, edge_index, batch, W1, b1, W2, b2, W3, b3, Wl, bl):
    raise NotImplementedError("write your pallas kernel here")


# speedup vs baseline: 9.7606x; 9.7606x over previous
import functools

import jax
import jax.numpy as jnp
from jax import lax
from jax.experimental import pallas as pl
from jax.experimental.pallas import tpu as pltpu
from jax.experimental.pallas import tpu_sc as plsc

N_PAD = 10240
E_PAD = 323584
NB = 1024
GRID = N_PAD // NB
NW = 32
EDGES_PER_W = E_PAD // NW
CHUNK = 128
CHUNKS_PER_W = EDGES_PER_W // CHUNK
ROWS_PER_TILE = N_PAD // 16
H = 128
G = 128

_MESH = plsc.VectorSubcoreMesh(core_axis_name="c", subcore_axis_name="s")


@functools.partial(
    pl.kernel,
    out_type=jax.ShapeDtypeStruct((NW, N_PAD), jnp.float32),
    mesh=_MESH,
    scratch_types=[
        pltpu.VMEM((EDGES_PER_W,), jnp.int32),
        pltpu.VMEM((N_PAD,), jnp.float32),
    ],
    compiler_params=pltpu.CompilerParams(needs_layout_passes=False),
)
def _sc_degree(dst_hbm, out_hbm, dst_v, deg_v):
    w = lax.axis_index("c") * 16 + lax.axis_index("s")
    base = pl.multiple_of(w * EDGES_PER_W, 8)
    pltpu.sync_copy(dst_hbm.at[pl.ds(base, EDGES_PER_W)], dst_v)

    zeros16 = jnp.zeros((16,), jnp.float32)

    def zero_body(i, _):
        deg_v[pl.ds(i * 16, 16)] = zeros16
        return 0

    lax.fori_loop(0, N_PAD // 16, zero_body, 0)

    ones16 = jnp.ones((16,), jnp.float32)

    def hist_body(i, _):
        d = dst_v[pl.ds(i * 16, 16)]
        plsc.addupdate_scatter(deg_v, [d], ones16)
        return 0

    lax.fori_loop(0, EDGES_PER_W // 16, hist_body, 0)
    pltpu.sync_copy(deg_v, out_hbm.at[w])


@functools.partial(
    pl.kernel,
    out_type=jax.ShapeDtypeStruct((2, N_PAD, H), jnp.float32),
    mesh=_MESH,
    scratch_types=[
        pltpu.VMEM_SHARED((N_PAD, H), jnp.float32),
        pltpu.VMEM((CHUNK,), jnp.int32),
        pltpu.VMEM((CHUNK,), jnp.int32),
        pltpu.VMEM((CHUNK, H), jnp.float32),
        pltpu.SemaphoreType.DMA,
    ],
)
def _sc_agg(y_hbm, src_hbm, dst_hbm, zb_hbm, out_hbm,
            agg_sh, src_v, dst_v, rows_v, sem):
    c = lax.axis_index("c")
    s = lax.axis_index("s")
    w = c * 16 + s

    pltpu.sync_copy(zb_hbm, agg_sh.at[pl.ds(s * ROWS_PER_TILE, ROWS_PER_TILE)])
    plsc.subcore_barrier()

    e0 = w * EDGES_PER_W

    def body(i, _):
        base = pl.multiple_of(e0 + i * CHUNK, 8)
        pltpu.sync_copy(src_hbm.at[pl.ds(base, CHUNK)], src_v)
        pltpu.sync_copy(dst_hbm.at[pl.ds(base, CHUNK)], dst_v)
        pltpu.async_copy(y_hbm.at[src_v], rows_v, sem).wait()
        pltpu.sync_copy(rows_v, agg_sh.at[dst_v], add=True)
        return 0

    lax.fori_loop(0, CHUNKS_PER_W, body, 0)
    plsc.subcore_barrier()

    r0 = s * ROWS_PER_TILE
    pltpu.sync_copy(agg_sh.at[pl.ds(r0, ROWS_PER_TILE)],
                    out_hbm.at[c, pl.ds(r0, ROWS_PER_TILE)])


def _dinv_of(degp_blk):
    return lax.rsqrt(1.0 + jnp.sum(degp_blk, axis=0))


def _tc_pre_body(x_ref, w_ref, degp_ref, y_ref):
    dinv = _dinv_of(degp_ref[...])
    xw = jnp.dot(x_ref[...], w_ref[...], preferred_element_type=jnp.float32)
    y_ref[...] = xw * dinv[:, None]


def _tc_mid_body(y_ref, p_ref, b_ref, degp_ref, w_ref, o_ref):
    dinv = _dinv_of(degp_ref[...])
    agg = y_ref[...] + p_ref[0] + p_ref[1]
    h = jnp.maximum(agg * dinv[:, None] + b_ref[...], 0.0)
    hw = jnp.dot(h, w_ref[...], preferred_element_type=jnp.float32)
    o_ref[...] = hw * dinv[:, None]


def _tc_last_body(y_ref, p_ref, b_ref, degp_ref, batch_ref, wl_ref, bl_ref,
                  o_ref, sums_acc, cnt_acc):
    i = pl.program_id(0)

    @pl.when(i == 0)
    def _():
        sums_acc[...] = jnp.zeros_like(sums_acc)
        cnt_acc[...] = jnp.zeros_like(cnt_acc)

    dinv = _dinv_of(degp_ref[...])
    agg = y_ref[...] + p_ref[0] + p_ref[1]
    h = jnp.maximum(agg * dinv[:, None] + b_ref[...], 0.0)

    gid = lax.broadcasted_iota(jnp.int32, (G, NB), 0)
    m = (batch_ref[...] == gid).astype(jnp.float32)
    sums_acc[...] += jnp.dot(m, h, preferred_element_type=jnp.float32)
    cnt_acc[...] += jnp.sum(m, axis=1, keepdims=True)

    @pl.when(i == GRID - 1)
    def _():
        pooled = sums_acc[...] / jnp.maximum(cnt_acc[...], 1.0)
        o_ref[...] = jnp.dot(pooled, wl_ref[...],
                             preferred_element_type=jnp.float32) + bl_ref[...]


_row_spec = pl.BlockSpec((NB, H), lambda i: (i, 0))
_p_spec = pl.BlockSpec((2, NB, H), lambda i: (0, i, 0))
_w_spec = pl.BlockSpec((H, H), lambda i: (0, 0))
_b_spec = pl.BlockSpec((1, H), lambda i: (0, 0))
_degp_spec = pl.BlockSpec((NW, NB), lambda i: (0, i))
_batch_spec = pl.BlockSpec((1, NB), lambda i: (0, i))

_tc_pre = pl.pallas_call(
    _tc_pre_body,
    grid=(GRID,),
    in_specs=[_row_spec, _w_spec, _degp_spec],
    out_specs=_row_spec,
    out_shape=jax.ShapeDtypeStruct((N_PAD, H), jnp.float32),
)

_tc_mid = pl.pallas_call(
    _tc_mid_body,
    grid=(GRID,),
    in_specs=[_row_spec, _p_spec, _b_spec, _degp_spec, _w_spec],
    out_specs=_row_spec,
    out_shape=jax.ShapeDtypeStruct((N_PAD, H), jnp.float32),
)

_tc_last = pl.pallas_call(
    _tc_last_body,
    grid=(GRID,),
    in_specs=[_row_spec, _p_spec, _b_spec, _degp_spec, _batch_spec,
              _w_spec, _b_spec],
    out_specs=pl.BlockSpec((G, H), lambda i: (0, 0)),
    out_shape=jax.ShapeDtypeStruct((G, H), jnp.float32),
    scratch_shapes=[pltpu.VMEM((G, H), jnp.float32),
                    pltpu.VMEM((G, 1), jnp.float32)],
)


def kernel(x, edge_index, batch, W1, b1, W2, b2, W3, b3, Wl, bl):
    n, f_in = x.shape
    e = edge_index.shape[1]
    c_out = Wl.shape[1]

    pad_node = jnp.full((E_PAD - e,), N_PAD - 1, jnp.int32)
    src = jnp.concatenate([edge_index[0], pad_node])
    dst = jnp.concatenate([edge_index[1], pad_node])

    x_pad = jnp.pad(x, ((0, N_PAD - n), (0, 0)))
    batch2d = jnp.pad(batch, (0, N_PAD - n),
                      constant_values=G).reshape(1, N_PAD)
    wl_pad = jnp.pad(Wl, ((0, 0), (0, H - c_out)))
    bl_pad = jnp.pad(bl, (0, H - c_out)).reshape(1, H)
    zb = jnp.zeros((ROWS_PER_TILE, H), jnp.float32)

    degp = _sc_degree(dst)

    y1 = _tc_pre(x_pad, W1, degp)
    p1 = _sc_agg(y1, src, dst, zb)
    y2 = _tc_mid(y1, p1, b1.reshape(1, H), degp, W2)
    p2 = _sc_agg(y2, src, dst, zb)
    y3 = _tc_mid(y2, p2, b2.reshape(1, H), degp, W3)
    p3 = _sc_agg(y3, src, dst, zb)
    out = _tc_last(y3, p3, b3.reshape(1, H), degp, batch2d, wl_pad, bl_pad)
    return out[:, :c_out]
